# baseline (speedup 1.0000x reference)
import functools

import jax
import jax.numpy as jnp
from jax import lax
from jax.experimental import pallas as pl
from jax.experimental.pallas import tpu as pltpu
from jax.experimental.pallas import tpu_sc as plsc

_BATCH = 4096
_HIST = 200
_D = 64
_V = 1000001
_NC = 2
_NS = 16
_NW = _NC * _NS
_BB = _BATCH // _NW
_L = 16
_NSLOT = 4

_NCT = 7812
_VMAIN = _NCT * 128
_TAIL = _V - _VMAIN
_KMAX = 246

_mesh = plsc.VectorSubcoreMesh(core_axis_name="c", subcore_axis_name="s")


@functools.partial(
    pl.kernel,
    mesh=_mesh,
    out_type=jax.ShapeDtypeStruct((_V * _D,), jnp.float32),
    scratch_types=[
        pltpu.VMEM((_D, 129), jnp.float32),
        pltpu.VMEM((_D, 129), jnp.float32),
        pltpu.VMEM((128 * _D,), jnp.float32),
        pltpu.VMEM((128 * _D,), jnp.float32),
        pltpu.VMEM((72 * 128,), jnp.float32),
        pltpu.SemaphoreType.DMA,
        pltpu.SemaphoreType.DMA,
    ],
    compiler_params=pltpu.CompilerParams(
        use_tc_tiling_on_sc=True, needs_layout_passes=False
    ),
)
def _tab_kernel(tabT_hbm, tail_hbm, flat_hbm, st0, st1, rw0, rw1, tail_v,
                sem0, sem1):
    wid = lax.axis_index("s") * _NC + lax.axis_index("c")
    stage = (st0, st1)
    rowsc = (rw0, rw1)
    sem = (sem0, sem1)

    iota = lax.iota(jnp.int32, _L)
    dids = [iota + c * _L for c in range(_D // _L)]

    def fire_reads(ct, b):
        for dt in range(8):
            pltpu.async_copy(
                tabT_hbm.at[pl.ds(8 * dt, 8), pl.ds(ct * 128, 128)],
                stage[b].at[pl.ds(8 * dt, 8), pl.ds(0, 128)],
                sem[b],
            )

    def drain_reads(b):
        for dt in range(8):
            pltpu.make_async_copy(
                tabT_hbm.at[pl.ds(0, 8), pl.ds(0, 128)],
                stage[b].at[pl.ds(8 * dt, 8), pl.ds(0, 128)],
                sem[b],
            ).wait()

    def transpose(b):
        src = stage[b]
        dst = rowsc[b]

        def cm_body(g, carry):
            for j in range(8):
                cm = 8 * g + j
                cm_vec = jnp.full((_L,), 0, jnp.int32) + cm
                for c in range(_D // _L):
                    v = plsc.load_gather(src, [dids[c], cm_vec])
                    dst[pl.ds(cm * _D + c * _L, _L)] = v
            return carry

        lax.fori_loop(0, 16, cm_body, 0)

    def body(u, carry):
        for bi in range(2):
            k = 2 * u + bi
            ct = wid + _NW * k

            @pl.when(ct < _NCT)
            def _():
                drain_reads(bi)

                @pl.when(ct + 2 * _NW < _NCT)
                def _():
                    fire_reads(ct + 2 * _NW, bi)

                transpose(bi)
                pltpu.sync_copy(
                    rowsc[bi], flat_hbm.at[pl.ds(ct * 128 * _D, 128 * _D)]
                )
        return carry

    fire_reads(wid, 0)

    @pl.when(wid + _NW < _NCT)
    def _():
        fire_reads(wid + _NW, 1)

    lax.fori_loop(0, _KMAX // 2, body, 0)

    @pl.when(wid == 0)
    def _():
        pltpu.sync_copy(tail_hbm, tail_v)

        def tail_body(r, carry):
            pltpu.sync_copy(
                tail_v.at[pl.ds(r * 128, _D)],
                flat_hbm.at[pl.ds((_VMAIN + r) * _D, _D)],
            )
            return carry

        lax.fori_loop(0, _TAIL, tail_body, 0)


@functools.partial(
    pl.kernel,
    mesh=_mesh,
    out_type=jax.ShapeDtypeStruct((_HIST, 8, _NW, 8, _BB), jnp.float32),
    scratch_types=[
        pltpu.VMEM((_HIST, _BB), jnp.int32),
        pltpu.VMEM((_BB, _D), jnp.float32),
        pltpu.VMEM((_BB, _D), jnp.float32),
        pltpu.VMEM((_BB, _D), jnp.float32),
        pltpu.VMEM((_BB, _D), jnp.float32),
        pltpu.VMEM((_D, _BB + 1), jnp.float32),
        pltpu.VMEM((_D, _BB + 1), jnp.float32),
        pltpu.SemaphoreType.DMA,
        pltpu.SemaphoreType.DMA,
        pltpu.SemaphoreType.DMA,
        pltpu.SemaphoreType.DMA,
        pltpu.SemaphoreType.DMA,
        pltpu.SemaphoreType.DMA,
    ],
    compiler_params=pltpu.CompilerParams(
        use_tc_tiling_on_sc=False, needs_layout_passes=False
    ),
)
def _emb_kernel(table_hbm, idx_hbm, out_hbm, idx_v, r0, r1, r2, r3, t0, t1,
                g0, g1, g2, g3, s0, s1):
    wid = lax.axis_index("s") * _NC + lax.axis_index("c")
    rows = (r0, r1, r2, r3)
    tbuf = (t0, t1)
    sem_g = (g0, g1, g2, g3)
    sem_s = (s0, s1)

    pltpu.sync_copy(idx_hbm.at[wid], idx_v)

    iota = lax.iota(jnp.int32, _L)
    dids = [iota + c * _L for c in range(_D // _L)]

    def fire_gather(i, slot):
        pltpu.async_copy(table_hbm.at[idx_v.at[i]], rows[slot], sem_g[slot])

    def drain_gather(slot):
        pltpu.make_async_copy(
            table_hbm.at[pl.ds(0, _BB)], rows[slot], sem_g[slot]
        ).wait()

    def drain_stores(b):
        for dt in range(8):
            pltpu.make_async_copy(
                out_hbm.at[0, 0, 0],
                tbuf[b].at[pl.ds(8 * dt, 8), pl.ds(0, _BB)],
                sem_s[b],
            ).wait()

    def transpose(slot, b):
        src = rows[slot]
        dst = tbuf[b]

        def bm_body(g, carry):
            for j in range(8):
                bm = 8 * g + j
                bm_vec = jnp.full((_L,), 0, jnp.int32) + bm
                for c in range(_D // _L):
                    v = src[bm, pl.ds(c * _L, _L)]
                    plsc.store_scatter(dst, [dids[c], bm_vec], v)
            return carry

        lax.fori_loop(0, _BB // 8, bm_body, 0)

    def fire_stores(i, b):
        for dt in range(8):
            pltpu.async_copy(
                tbuf[b].at[pl.ds(8 * dt, 8), pl.ds(0, _BB)],
                out_hbm.at[i, dt, wid],
                sem_s[b],
            )

    for k in range(_NSLOT):
        fire_gather(k, k)

    def body(u, carry):
        for k in range(_NSLOT):
            i = _NSLOT * u + k
            b = k % 2
            drain_gather(k)

            @pl.when(i >= 2)
            def _():
                drain_stores(b)

            transpose(k, b)
            fire_stores(i, b)

            @pl.when(i + _NSLOT < _HIST)
            def _():
                fire_gather(i + _NSLOT, k)
        return carry

    lax.fori_loop(0, _HIST // _NSLOT, body, 0)
    drain_stores(0)
    drain_stores(1)


def kernel(item_seq, ID_embeddings):
    tabT = ID_embeddings.T
    tail = jnp.pad(
        ID_embeddings[_VMAIN:], ((0, 72 - _TAIL), (0, 128 - _D))
    ).reshape(-1)
    flat = _tab_kernel(tabT, tail)
    table = flat.reshape(_V, _D)

    idx = (
        item_seq.astype(jnp.int32)
        .reshape(_NW, _BB, _HIST)
        .transpose(0, 2, 1)
    )
    five = _emb_kernel(table, idx)
    return five.transpose(2, 4, 0, 1, 3).reshape(_BATCH, _HIST, _D)


# speedup vs baseline: 1.9985x; 1.9985x over previous
import functools

import jax
import jax.numpy as jnp
from jax import lax
from jax.experimental import pallas as pl
from jax.experimental.pallas import tpu as pltpu
from jax.experimental.pallas import tpu_sc as plsc

_BATCH = 4096
_HIST = 200
_D = 64
_NC = 2
_NS = 16
_NW = _NC * _NS
_BB = _BATCH // _NW
_L = 16
_NSLOT = 4

_mesh = plsc.VectorSubcoreMesh(core_axis_name="c", subcore_axis_name="s")


@functools.partial(
    pl.kernel,
    mesh=_mesh,
    out_type=jax.ShapeDtypeStruct((_HIST, 8, _NW, 8, _BB), jnp.float32),
    scratch_types=[
        pltpu.VMEM((_HIST, _BB), jnp.int32),
        pltpu.VMEM((_BB, _D), jnp.float32),
        pltpu.VMEM((_BB, _D), jnp.float32),
        pltpu.VMEM((_BB, _D), jnp.float32),
        pltpu.VMEM((_BB, _D), jnp.float32),
        pltpu.VMEM((_D, _BB + 1), jnp.float32),
        pltpu.VMEM((_D, _BB + 1), jnp.float32),
        pltpu.SemaphoreType.DMA,
        pltpu.SemaphoreType.DMA,
        pltpu.SemaphoreType.DMA,
        pltpu.SemaphoreType.DMA,
        pltpu.SemaphoreType.DMA,
        pltpu.SemaphoreType.DMA,
    ],
    compiler_params=pltpu.CompilerParams(
        use_tc_tiling_on_sc=False, needs_layout_passes=False
    ),
)
def _emb_kernel(table_hbm, idx_hbm, out_hbm, idx_v, r0, r1, r2, r3, t0, t1,
                g0, g1, g2, g3, s0, s1):
    wid = lax.axis_index("s") * _NC + lax.axis_index("c")
    rows = (r0, r1, r2, r3)
    tbuf = (t0, t1)
    sem_g = (g0, g1, g2, g3)
    sem_s = (s0, s1)

    pltpu.sync_copy(idx_hbm.at[wid], idx_v)

    iota = lax.iota(jnp.int32, _L)
    dids = [iota + c * _L for c in range(_D // _L)]

    def fire_gather(i, slot):
        pltpu.async_copy(table_hbm.at[idx_v.at[i]], rows[slot], sem_g[slot])

    def drain_gather(slot):
        pltpu.make_async_copy(
            table_hbm.at[pl.ds(0, _BB)], rows[slot], sem_g[slot]
        ).wait()

    def drain_stores(b):
        for dt in range(8):
            pltpu.make_async_copy(
                out_hbm.at[0, 0, 0],
                tbuf[b].at[pl.ds(8 * dt, 8), pl.ds(0, _BB)],
                sem_s[b],
            ).wait()

    def transpose(slot, b):
        src = rows[slot]
        dst = tbuf[b]

        def bm_body(g, carry):
            for j in range(8):
                bm = 8 * g + j
                bm_vec = jnp.full((_L,), 0, jnp.int32) + bm
                for c in range(_D // _L):
                    v = src[bm, pl.ds(c * _L, _L)]
                    plsc.store_scatter(dst, [dids[c], bm_vec], v)
            return carry

        lax.fori_loop(0, _BB // 8, bm_body, 0)

    def fire_stores(i, b):
        for dt in range(8):
            pltpu.async_copy(
                tbuf[b].at[pl.ds(8 * dt, 8), pl.ds(0, _BB)],
                out_hbm.at[i, dt, wid],
                sem_s[b],
            )

    for k in range(_NSLOT):
        fire_gather(k, k)

    def body(u, carry):
        for k in range(_NSLOT):
            i = _NSLOT * u + k
            b = k % 2
            drain_gather(k)

            @pl.when(i >= 2)
            def _():
                drain_stores(b)

            transpose(k, b)
            fire_stores(i, b)

            @pl.when(i + _NSLOT < _HIST)
            def _():
                fire_gather(i + _NSLOT, k)
        return carry

    lax.fori_loop(0, _HIST // _NSLOT, body, 0)
    drain_stores(0)
    drain_stores(1)


def kernel(item_seq, ID_embeddings):
    idx = (
        item_seq.astype(jnp.int32)
        .reshape(_NW, _BB, _HIST)
        .transpose(0, 2, 1)
    )
    five = _emb_kernel(ID_embeddings, idx)
    return five.transpose(2, 4, 0, 1, 3).reshape(_BATCH, _HIST, _D)
